# baseline (speedup 1.0000x reference)
import functools

import jax
import jax.numpy as jnp
from jax import lax
from jax.experimental import pallas as pl
from jax.experimental.pallas import tpu as pltpu
from jax.experimental.pallas import tpu_sc as plsc

_N = 10000
_E = 160000
_K = 4
_H = 64
_NP = 10240
_RPT = 640
_EPAD = 163840
_CHUNK = 128


def _make_aggregate(edge_split):
  mesh = plsc.VectorSubcoreMesh(core_axis_name="c", subcore_axis_name="s")
  nch = (_EPAD // 32 if edge_split else _EPAD // 16) // _CHUNK

  @functools.partial(
      pl.kernel,
      mesh=mesh,
      out_type=jax.ShapeDtypeStruct((2 * _NP, 128), jnp.float32),
      scratch_types=[
          pltpu.VMEM((2, _CHUNK), jnp.int32),
          pltpu.VMEM((2, _CHUNK), jnp.int32),
          pltpu.VMEM((_CHUNK, 128), jnp.float32),
          pltpu.VMEM((_CHUNK, 128), jnp.float32),
          pltpu.VMEM_SHARED((_NP, 128), jnp.float32),
          pltpu.SemaphoreType.DMA,
          pltpu.SemaphoreType.DMA,
          pltpu.SemaphoreType.DMA,
          pltpu.SemaphoreType.DMA,
          pltpu.SemaphoreType.DMA,
      ],
  )
  def agg(hs, srcp, dstp, out, srcv, dstv, st0, st1, acc, gsem, ssem,
          xsem, dsem0, dsem1):
    c = lax.axis_index("c")
    s = lax.axis_index("s")
    r0 = s * _RPT
    rbase = (c * 16 + s) * nch if edge_split else s * nch
    bias = c * _NP
    init_base = (2 * c if edge_split else c) * _NP
    ci = pltpu.async_copy(hs.at[pl.ds(init_base + r0, _RPT)],
                          acc.at[pl.ds(r0, _RPT)], xsem)
    stages = (st0, st1)
    dsems = (dsem0, dsem1)

    def bias_src(bi):
      for j in range(_CHUNK // 16):
        srcv[bi, pl.ds(j * 16, 16)] = srcv[bi, pl.ds(j * 16, 16)] + bias

    def gath(bi, buf):
      return pltpu.async_copy(hs.at[srcv.at[bi]], buf, gsem)

    def scat(bi, buf):
      return pltpu.async_copy(buf, acc.at[dstv.at[bi]], ssem, add=True)

    pltpu.sync_copy(srcp.at[rbase], srcv.at[0])
    pltpu.sync_copy(dstp.at[rbase], dstv.at[0])
    bias_src(0)
    ci.wait()
    plsc.subcore_barrier()

    g0 = gath(0, st0)
    cx = pltpu.async_copy(srcp.at[rbase + 1], srcv.at[1], xsem)
    pltpu.async_copy(dstp.at[rbase + 1], dstv.at[1], dsem1)
    cx.wait()
    bias_src(1)
    gath(1, st1)
    g0.wait()
    scat(0, st0)

    def body(j, carry):
      for b, i in ((1, 2 * j - 1), (0, 2 * j)):
        cx2 = pltpu.async_copy(srcp.at[rbase + i + 1], srcv.at[1 - b], xsem)
        pltpu.make_async_copy(stages[1 - b], acc.at[dstv.at[1 - b]],
                              ssem).wait()
        pltpu.async_copy(dstp.at[rbase + i + 1], dstv.at[1 - b],
                         dsems[1 - b])
        cx2.wait()
        bias_src(1 - b)
        gath(1 - b, stages[1 - b])
        pltpu.make_async_copy(hs.at[srcv.at[b]], stages[b], gsem).wait()
        pltpu.make_async_copy(dstp.at[rbase + i], dstv.at[b],
                              dsems[b]).wait()
        scat(b, stages[b])
      return carry

    lax.fori_loop(1, nch // 2, body, 0)

    pltpu.make_async_copy(st0, acc.at[dstv.at[0]], ssem).wait()
    pltpu.make_async_copy(hs.at[srcv.at[1]], st1, gsem).wait()
    pltpu.make_async_copy(dstp.at[rbase + nch - 1], dstv.at[1],
                          dsems[1]).wait()
    scat(1, st1).wait()
    plsc.subcore_barrier()
    pltpu.sync_copy(acc.at[pl.ds(r0, _RPT)],
                    out.at[pl.ds(c * _NP + r0, _RPT)])

  return agg


def _make_mlp(in_widths, in_offsets, layer_dims, relu_flags, out_widths,
              bf16_flags=None, bn=512):
  n_in = len(in_widths)
  nl = len(layer_dims)
  if bf16_flags is None:
    bf16_flags = [True] * nl

  def _dot(a, b, cast):
    if cast:
      return jnp.dot(a.astype(jnp.bfloat16), b.astype(jnp.bfloat16),
                     preferred_element_type=jnp.float32)
    return jnp.dot(a, b, preferred_element_type=jnp.float32,
                   precision=lax.Precision.HIGHEST)

  def body(*refs):
    z_refs = refs[:n_in]
    w_refs = refs[n_in:n_in + nl]
    b_refs = refs[n_in + nl:n_in + 2 * nl]
    out_refs = refs[n_in + 2 * nl:]
    ofs = 0
    acc = None
    w0 = w_refs[0][...]
    for zi, zr in enumerate(z_refs):
      part = _dot(zr[...], w0[ofs:ofs + in_widths[zi], :], bf16_flags[0])
      acc = part if acc is None else acc + part
      ofs += in_widths[zi]
    acc = acc + b_refs[0][...]
    if relu_flags[0]:
      acc = jnp.maximum(acc, 0.0)
    for i in range(1, nl):
      acc = _dot(acc, w_refs[i][...], bf16_flags[i])
      acc = acc + b_refs[i][...]
      if relu_flags[i]:
        acc = jnp.maximum(acc, 0.0)
    ofs = 0
    for oi, oref in enumerate(out_refs):
      oref[...] = acc[:, ofs:ofs + out_widths[oi]]
      ofs += out_widths[oi]

  in_specs = [
      pl.BlockSpec((bn, w), lambda i, o=o: (i + o, 0))
      for w, o in zip(in_widths, in_offsets)
  ]
  for (a, b) in layer_dims:
    in_specs.append(pl.BlockSpec((a, b), lambda i: (0, 0)))
  for (_, b) in layer_dims:
    in_specs.append(pl.BlockSpec((1, b), lambda i: (0, 0)))
  out_specs = [pl.BlockSpec((bn, w), lambda i: (i, 0)) for w in out_widths]
  out_shape = [jax.ShapeDtypeStruct((_NP, w), jnp.float32) for w in out_widths]
  if len(out_widths) == 1:
    out_specs, out_shape = out_specs[0], out_shape[0]
  return pl.pallas_call(
      body,
      grid=(_NP // bn,),
      in_specs=in_specs,
      out_specs=out_specs,
      out_shape=out_shape,
  )


def kernel(x, edge_index,
           enc0_W1, enc0_b1, enc0_W2, enc0_b2,
           enc1_W1, enc1_b1, enc1_W2, enc1_b2,
           enc2_W1, enc2_b1, enc2_W2, enc2_b2,
           rho_W0, rho_b0, rho_W1, rho_b1, rho_W2, rho_b2):
  f32 = jnp.float32
  src = edge_index[0]
  dst = edge_index[1]
  eye = jnp.eye(_K, dtype=f32)

  W1e = jnp.zeros((128, _K * _H), f32).at[:_K, :].set(jnp.kron(eye, enc0_W1))
  W1e2 = jnp.concatenate([W1e, W1e], axis=0)
  b1t0 = jnp.tile(enc0_b1, _K).reshape(1, -1)
  BD2_0 = jnp.kron(eye, enc0_W2)
  b2t0 = jnp.tile(enc0_b2, _K).reshape(1, -1)
  BD1_1 = jnp.kron(eye, enc1_W1)
  b1t1 = jnp.tile(enc1_b1, _K).reshape(1, -1)
  BD2_1 = jnp.kron(eye, enc1_W2)
  b2t1 = jnp.tile(enc1_b2, _K).reshape(1, -1)
  BD1_2 = jnp.kron(eye, enc2_W1)
  b1t2 = jnp.tile(enc2_b1, _K).reshape(1, -1)
  BD2_2 = jnp.kron(eye, enc2_W2)
  b2t2 = jnp.tile(enc2_b2, _K).reshape(1, -1)
  W0p = jnp.zeros((_K * _H, 128), f32).at[:, :_H].set(rho_W0)
  b0p = jnp.zeros((1, 128), f32).at[0, :_H].set(rho_b0)
  W1p = jnp.zeros((128, 128), f32).at[:_H, :_H].set(rho_W1)
  b1p = jnp.zeros((1, 128), f32).at[0, :_H].set(rho_b1)
  W2p = jnp.zeros((128, 128), f32).at[:_H, :_K].set(rho_W2)
  b2p = jnp.zeros((1, 128), f32).at[0, :_K].set(rho_b2)

  srcp = jnp.concatenate(
      [src, jnp.zeros((_EPAD - _E,), jnp.int32)]).reshape(-1, _CHUNK)
  dstp = jnp.concatenate(
      [dst, jnp.full((_EPAD - _E,), _NP - 1, jnp.int32)]).reshape(-1, _CHUNK)
  x2 = x.reshape(_N, _K)
  hs0 = (jnp.zeros((3 * _NP, 128), f32)
         .at[:_N, :_K].set(x2)
         .at[_NP:_NP + _N, :_K].set(x2))

  agg_e = _make_aggregate(True)
  agg_c = _make_aggregate(False)
  nb = _NP // 512
  mlp0 = _make_mlp([128, 128], [0, nb], [(256, 256), (256, 256)],
                   [True, True], [128, 128], bf16_flags=[False, True])
  mlp1 = _make_mlp([128, 128], [0, nb], [(256, 256), (256, 256)],
                   [True, True], [128, 128])
  mlp2 = _make_mlp([128, 128], [0, nb],
                   [(256, 256), (256, 256), (256, 128), (128, 128),
                    (128, 128)],
                   [True, False, True, True, False], [128])

  p = agg_e(hs0, srcp, dstp)
  h1a, h1b = mlp0(p, p, W1e2, BD2_0, b1t0, b2t0)
  hs1 = jnp.concatenate([h1a, h1b], axis=0)
  zs1 = agg_c(hs1, srcp, dstp)
  h2a, h2b = mlp1(zs1, zs1, BD1_1, BD2_1, b1t1, b2t1)
  hs2 = jnp.concatenate([h2a, h2b], axis=0)
  zs2 = agg_c(hs2, srcp, dstp)
  out = mlp2(zs2, zs2, BD1_2, BD2_2, W0p, W1p, W2p, b1t2, b2t2, b0p, b1p,
             b2p)
  return out[:_N, :_K].reshape(_N, _K, 1)


# speedup vs baseline: 1.1009x; 1.1009x over previous
import functools

import jax
import jax.numpy as jnp
from jax import lax
from jax.experimental import pallas as pl
from jax.experimental.pallas import tpu as pltpu
from jax.experimental.pallas import tpu_sc as plsc

_N = 10000
_E = 160000
_K = 4
_H = 64
_NP = 10240
_RPT = 640
_EPAD = 163840
_CHUNK = 128


def _make_aggregate(edge_split):
  mesh = plsc.VectorSubcoreMesh(core_axis_name="c", subcore_axis_name="s")
  nch = (_EPAD // 32 if edge_split else _EPAD // 16) // _CHUNK

  @functools.partial(
      pl.kernel,
      mesh=mesh,
      out_type=jax.ShapeDtypeStruct((2 * _NP, 128), jnp.float32),
      scratch_types=[
          pltpu.VMEM((2, _CHUNK), jnp.int32),
          pltpu.VMEM((2, _CHUNK), jnp.int32),
          pltpu.VMEM((_CHUNK, 128), jnp.float32),
          pltpu.VMEM((_CHUNK, 128), jnp.float32),
          pltpu.VMEM_SHARED((_NP, 128), jnp.float32),
          pltpu.SemaphoreType.DMA,
          pltpu.SemaphoreType.DMA,
          pltpu.SemaphoreType.DMA,
          pltpu.SemaphoreType.DMA,
          pltpu.SemaphoreType.DMA,
      ],
  )
  def agg(hs, srcp, dstp, out, srcv, dstv, st0, st1, acc, gsem, ssem,
          xsem, dsem0, dsem1):
    c = lax.axis_index("c")
    s = lax.axis_index("s")
    r0 = s * _RPT
    rbase = (c * 16 + s) * nch if edge_split else s * nch
    bias = c * _NP
    init_base = c * _NP
    ci = pltpu.async_copy(hs.at[pl.ds(init_base + r0, _RPT)],
                          acc.at[pl.ds(r0, _RPT)], xsem)
    stages = (st0, st1)
    dsems = (dsem0, dsem1)

    def bias_src(bi):
      if not edge_split:
        for j in range(_CHUNK // 16):
          srcv[bi, pl.ds(j * 16, 16)] = srcv[bi, pl.ds(j * 16, 16)] + bias

    def gath(bi, buf):
      return pltpu.async_copy(hs.at[srcv.at[bi]], buf, gsem)

    def scat(bi, buf):
      return pltpu.async_copy(buf, acc.at[dstv.at[bi]], ssem, add=True)

    pltpu.sync_copy(srcp.at[rbase], srcv.at[0])
    pltpu.sync_copy(dstp.at[rbase], dstv.at[0])
    bias_src(0)
    ci.wait()
    plsc.subcore_barrier()

    g0 = gath(0, st0)
    cx = pltpu.async_copy(srcp.at[rbase + 1], srcv.at[1], xsem)
    pltpu.async_copy(dstp.at[rbase + 1], dstv.at[1], dsem1)
    cx.wait()
    bias_src(1)
    gath(1, st1)
    g0.wait()
    scat(0, st0)

    def body(j, carry):
      for b, i in ((1, 2 * j - 1), (0, 2 * j)):
        cx2 = pltpu.async_copy(srcp.at[rbase + i + 1], srcv.at[1 - b], xsem)
        pltpu.make_async_copy(stages[1 - b], acc.at[dstv.at[1 - b]],
                              ssem).wait()
        pltpu.async_copy(dstp.at[rbase + i + 1], dstv.at[1 - b],
                         dsems[1 - b])
        cx2.wait()
        bias_src(1 - b)
        gath(1 - b, stages[1 - b])
        pltpu.make_async_copy(hs.at[srcv.at[b]], stages[b], gsem).wait()
        pltpu.make_async_copy(dstp.at[rbase + i], dstv.at[b],
                              dsems[b]).wait()
        scat(b, stages[b])
      return carry

    lax.fori_loop(1, nch // 2, body, 0)

    pltpu.make_async_copy(st0, acc.at[dstv.at[0]], ssem).wait()
    pltpu.make_async_copy(hs.at[srcv.at[1]], st1, gsem).wait()
    pltpu.make_async_copy(dstp.at[rbase + nch - 1], dstv.at[1],
                          dsems[1]).wait()
    scat(1, st1).wait()
    plsc.subcore_barrier()
    pltpu.sync_copy(acc.at[pl.ds(r0, _RPT)],
                    out.at[pl.ds(c * _NP + r0, _RPT)])

  return agg


def _make_mlp(in_widths, in_offsets, layer_dims, relu_flags, out_widths,
              bf16_flags=None, bn=512):
  n_in = len(in_widths)
  nl = len(layer_dims)
  if bf16_flags is None:
    bf16_flags = [True] * nl

  def _dot(a, b, cast):
    if cast:
      return jnp.dot(a.astype(jnp.bfloat16), b.astype(jnp.bfloat16),
                     preferred_element_type=jnp.float32)
    return jnp.dot(a, b, preferred_element_type=jnp.float32,
                   precision=lax.Precision.HIGHEST)

  def body(*refs):
    z_refs = refs[:n_in]
    w_refs = refs[n_in:n_in + nl]
    b_refs = refs[n_in + nl:n_in + 2 * nl]
    out_refs = refs[n_in + 2 * nl:]
    ofs = 0
    acc = None
    w0 = w_refs[0][...]
    for zi, zr in enumerate(z_refs):
      part = _dot(zr[...], w0[ofs:ofs + in_widths[zi], :], bf16_flags[0])
      acc = part if acc is None else acc + part
      ofs += in_widths[zi]
    acc = acc + b_refs[0][...]
    if relu_flags[0]:
      acc = jnp.maximum(acc, 0.0)
    for i in range(1, nl):
      acc = _dot(acc, w_refs[i][...], bf16_flags[i])
      acc = acc + b_refs[i][...]
      if relu_flags[i]:
        acc = jnp.maximum(acc, 0.0)
    ofs = 0
    for oi, oref in enumerate(out_refs):
      oref[...] = acc[:, ofs:ofs + out_widths[oi]]
      ofs += out_widths[oi]

  in_specs = [
      pl.BlockSpec((bn, w), lambda i, o=o: (i + o, 0))
      for w, o in zip(in_widths, in_offsets)
  ]
  for (a, b) in layer_dims:
    in_specs.append(pl.BlockSpec((a, b), lambda i: (0, 0)))
  for (_, b) in layer_dims:
    in_specs.append(pl.BlockSpec((1, b), lambda i: (0, 0)))
  out_specs = [pl.BlockSpec((bn, w), lambda i: (i, 0)) for w in out_widths]
  out_shape = [jax.ShapeDtypeStruct((_NP, w), jnp.float32) for w in out_widths]
  if len(out_widths) == 1:
    out_specs, out_shape = out_specs[0], out_shape[0]
  return pl.pallas_call(
      body,
      grid=(_NP // bn,),
      in_specs=in_specs,
      out_specs=out_specs,
      out_shape=out_shape,
  )


def kernel(x, edge_index,
           enc0_W1, enc0_b1, enc0_W2, enc0_b2,
           enc1_W1, enc1_b1, enc1_W2, enc1_b2,
           enc2_W1, enc2_b1, enc2_W2, enc2_b2,
           rho_W0, rho_b0, rho_W1, rho_b1, rho_W2, rho_b2):
  f32 = jnp.float32
  src = edge_index[0]
  dst = edge_index[1]
  eye = jnp.eye(_K, dtype=f32)

  W1e = jnp.zeros((128, _K * _H), f32).at[:_K, :].set(jnp.kron(eye, enc0_W1))
  W1e2 = jnp.concatenate([W1e, W1e], axis=0)
  b1t0 = jnp.tile(enc0_b1, _K).reshape(1, -1)
  BD2_0 = jnp.kron(eye, enc0_W2)
  b2t0 = jnp.tile(enc0_b2, _K).reshape(1, -1)
  BD1_1 = jnp.kron(eye, enc1_W1)
  b1t1 = jnp.tile(enc1_b1, _K).reshape(1, -1)
  BD2_1 = jnp.kron(eye, enc1_W2)
  b2t1 = jnp.tile(enc1_b2, _K).reshape(1, -1)
  BD1_2 = jnp.kron(eye, enc2_W1)
  b1t2 = jnp.tile(enc2_b1, _K).reshape(1, -1)
  BD2_2 = jnp.kron(eye, enc2_W2)
  b2t2 = jnp.tile(enc2_b2, _K).reshape(1, -1)
  W0p = jnp.zeros((_K * _H, 128), f32).at[:, :_H].set(rho_W0)
  b0p = jnp.zeros((1, 128), f32).at[0, :_H].set(rho_b0)
  W1p = jnp.zeros((128, 128), f32).at[:_H, :_H].set(rho_W1)
  b1p = jnp.zeros((1, 128), f32).at[0, :_H].set(rho_b1)
  W2p = jnp.zeros((128, 128), f32).at[:_H, :_K].set(rho_W2)
  b2p = jnp.zeros((1, 128), f32).at[0, :_K].set(rho_b2)

  srcp = jnp.concatenate(
      [src, jnp.zeros((_EPAD - _E,), jnp.int32)]).reshape(-1, _CHUNK)
  dstp = jnp.concatenate(
      [dst, jnp.full((_EPAD - _E,), _NP - 1, jnp.int32)]).reshape(-1, _CHUNK)
  hs0 = jnp.zeros((2 * _NP, 128), f32).at[:_N, :_K].set(x.reshape(_N, _K))

  agg_e = _make_aggregate(True)
  agg_c = _make_aggregate(False)
  nb = _NP // 512
  mlp0 = _make_mlp([128, 128], [0, nb], [(256, 256), (256, 256)],
                   [True, True], [128, 128], bf16_flags=[False, True])
  mlp1 = _make_mlp([128, 128], [0, nb], [(256, 256), (256, 256)],
                   [True, True], [128, 128])
  mlp2 = _make_mlp([128, 128], [0, nb],
                   [(256, 256), (256, 256), (256, 128), (128, 128),
                    (128, 128)],
                   [True, False, True, True, False], [128])

  p = agg_e(hs0, srcp, dstp)
  h1a, h1b = mlp0(p, p, W1e2, BD2_0, b1t0, b2t0)
  hs1 = jnp.concatenate([h1a, h1b], axis=0)
  zs1 = agg_c(hs1, srcp, dstp)
  h2a, h2b = mlp1(zs1, zs1, BD1_1, BD2_1, b1t1, b2t1)
  hs2 = jnp.concatenate([h2a, h2b], axis=0)
  zs2 = agg_c(hs2, srcp, dstp)
  out = mlp2(zs2, zs2, BD1_2, BD2_2, W0p, W1p, W2p, b1t2, b2t2, b0p, b1p,
             b2p)
  return out[:_N, :_K].reshape(_N, _K, 1)
